# baseline (speedup 1.0000x reference)
import math

import jax
import jax.numpy as jnp
from jax import lax
from jax.experimental import pallas as pl
from jax.experimental.pallas import tpu as pltpu
from jax.experimental.pallas import tpu_sc as plsc

_B = 1024
_V = 100000
_LS = 0.1
_PAD = 0
_CONF = 1.0 - _LS
_SMOOTH = _LS / (_V - 2)
_CONST = _LS * math.log(_SMOOTH) + _CONF * math.log(_CONF)

_CUT = 0
_BB = 32
_NC = 2
_NSUB = 16
_NW = _NC * _NSUB
_RPW = (_B - _CUT) // _NW
_NGRP = _RPW // 8
_TAIL0 = 99968
_TAILW = _V - _TAIL0
_NCHD = 13
_TPCF = 63
_TPCL = 25
_CWF = _TPCF * 128
_CWL = _TPCL * 128 + 32


def _tc_kernel(tgt_ref, x_ref, loss_ref):
    i = pl.program_id(0)
    x = x_ref[...]
    row_sum = jnp.sum(x, axis=1, keepdims=True)

    lane = lax.broadcasted_iota(jnp.int32, (1, 128), 1)
    sel_rows = []
    t_rows = []
    for r in range(_BB):
        t_r = tgt_ref[i * _BB + r]
        base = (t_r // 128) * 128
        chunk = x_ref[r:r + 1, pl.ds(base, 128)]
        sel_rows.append(jnp.where(lane == t_r - base, chunk, 0.0))
        t_rows.append(jnp.full((1, 1), t_r, dtype=jnp.int32))
    o_t = jnp.sum(jnp.concatenate(sel_rows, axis=0), axis=1, keepdims=True)
    t_vec = jnp.concatenate(t_rows, axis=0)

    o_pad = x[:, _PAD:_PAD + 1]
    loss = _SMOOTH * row_sum + (_CONF - _SMOOTH) * o_t - _SMOOTH * o_pad - _CONST
    loss_ref[...] = jnp.where(t_vec == _PAD, 0.0, loss)


def _tc_call(tgt, output):
    grid_spec = pltpu.PrefetchScalarGridSpec(
        num_scalar_prefetch=1,
        grid=(_CUT // _BB,),
        in_specs=[pl.BlockSpec((_BB, _V), lambda i, t: (i, 0))],
        out_specs=pl.BlockSpec((_BB, 1), lambda i, t: (i, 0)),
    )
    return pl.pallas_call(
        _tc_kernel,
        grid_spec=grid_spec,
        out_shape=jax.ShapeDtypeStruct((_CUT, 1), jnp.float32),
    )(tgt, output)


def _sc_body(out_ref, tgt_ref, loss_ref,
             buf0, buf1, tgtv, otv, opv, matv, lossv, sem0, sem1):
    wid = lax.axis_index("s") * _NC + lax.axis_index("c")
    row0 = _CUT + wid * _RPW
    lane = lax.iota(jnp.int32, 16)
    zero16f = jnp.zeros((16,), jnp.float32)
    zero16i = jnp.zeros((16,), jnp.int32)

    pltpu.sync_copy(tgt_ref.at[pl.ds(row0, _RPW)], tgtv)
    for hh in range(_RPW // 16):
        otv[pl.ds(hh * 16, 16)] = zero16f
        opv[pl.ds(hh * 16, 16)] = zero16f

    bufs = (buf0, buf1)
    sems = (sem0, sem1)
    ntot = _NGRP * _NCHD

    def _dma(j, b):
        g = j // _NCHD
        c = j - g * _NCHD
        r8 = row0 + g * 8
        full = pltpu.make_async_copy(
            out_ref.at[pl.ds(r8, 8), pl.ds(c * _CWF, _CWF)], bufs[b], sems[b])
        small = pltpu.make_async_copy(
            out_ref.at[pl.ds(r8, 8), pl.ds((_NCHD - 1) * _CWF, _CWL)],
            bufs[b].at[:, pl.ds(0, _CWL)], sems[b])
        return full, small, c == _NCHD - 1

    def _fire(j, b):
        full, small, is_last = _dma(j, b)

        @pl.when(is_last)
        def _():
            small.start()

        @pl.when(jnp.logical_not(is_last))
        def _():
            full.start()

    def _wait(j, b):
        full, small, is_last = _dma(j, b)

        @pl.when(is_last)
        def _():
            small.wait()

        @pl.when(jnp.logical_not(is_last))
        def _():
            full.wait()

    _fire(0, 0)
    _fire(1, 1)

    zeros8 = tuple(zero16f for _ in range(8))

    def pair_body(jj, accs):
        for b in range(2):
            j = jj * 2 + b
            g = j // _NCHD
            c = j - g * _NCHD
            is_last = c == _NCHD - 1
            _wait(j, b)
            buf = bufs[b]
            nm = jnp.where(is_last, _TPCL, _TPCF)

            def mbody(m, accs):
                out = []
                for a in range(8):
                    acc = accs[a]
                    for l0 in range(8):
                        acc = acc + buf[a, pl.ds(m * 128 + l0 * 16, 16)]
                    out.append(acc)
                return tuple(out)

            accs = lax.fori_loop(0, nm, mbody, accs)
            accs = tuple(
                a + jnp.where(is_last,
                              buf[i, pl.ds(_TPCL * 128, 16)]
                              + buf[i, pl.ds(_TPCL * 128 + 16, 16)], 0.0)
                for i, a in enumerate(accs))

            gm = g - (g // 2) * 2
            hoff = (g // 2) * 16
            t16 = tgtv[pl.ds(hoff, 16)]
            rel = t16 - c * _CWF
            cw = jnp.where(is_last, _CWL, _CWF)
            in_grp = (lane >= gm * 8) & (lane < gm * 8 + 8)
            cond = in_grp & (rel >= 0) & (rel < cw)
            a16 = jnp.clip(lane - gm * 8, 0, 7)
            c16 = jnp.clip(rel, 0, cw - 1)
            gt = plsc.load_gather(buf, [a16, c16])
            otv[pl.ds(hoff, 16)] = jnp.where(cond, gt, otv[pl.ds(hoff, 16)])
            gp = plsc.load_gather(buf, [a16, zero16i])
            opv[pl.ds(hoff, 16)] = jnp.where(
                in_grp & (c == 0), gp, opv[pl.ds(hoff, 16)])

            @pl.when(j + 2 < ntot)
            def _():
                _fire(j + 2, b)

            @pl.when(is_last)
            def _():
                for a in range(8):
                    matv[pl.ds((g * 8 + a) * 16, 16)] = accs[a]

            accs = tuple(jnp.where(is_last, 0.0, a) for a in accs)
        return accs

    lax.fori_loop(0, ntot // 2, pair_body, zeros8)

    for hh in range(_RPW // 16):
        s16 = zero16f
        for q in range(16):
            s16 = s16 + plsc.load_gather(matv, [(hh * 16 + lane) * 16 + q])
        t16 = tgtv[pl.ds(hh * 16, 16)]
        loss = (_SMOOTH * s16 + (_CONF - _SMOOTH) * otv[pl.ds(hh * 16, 16)]
                - _SMOOTH * opv[pl.ds(hh * 16, 16)] - _CONST)
        lossv[pl.ds(hh * 16, 16)] = jnp.where(t16 == _PAD, 0.0, loss)
    pltpu.sync_copy(lossv, loss_ref.at[pl.ds(wid * _RPW, _RPW)])


def _sc_call(output, tgt):
    mesh = plsc.VectorSubcoreMesh(core_axis_name="c", subcore_axis_name="s")
    fn = pl.kernel(
        _sc_body,
        out_type=jax.ShapeDtypeStruct((_B - _CUT,), jnp.float32),
        mesh=mesh,
        scratch_types=[
            pltpu.VMEM((8, _CWF), jnp.float32),
            pltpu.VMEM((8, _CWF), jnp.float32),
            pltpu.VMEM((_RPW,), jnp.int32),
            pltpu.VMEM((_RPW,), jnp.float32),
            pltpu.VMEM((_RPW,), jnp.float32),
            pltpu.VMEM((_RPW * 16,), jnp.float32),
            pltpu.VMEM((_RPW,), jnp.float32),
            pltpu.SemaphoreType.DMA,
            pltpu.SemaphoreType.DMA,
        ],
        compiler_params=pltpu.CompilerParams(
            needs_layout_passes=False, use_tc_tiling_on_sc=False),
    )
    return fn(output, tgt)


def kernel(output, target):
    tgt = target.astype(jnp.int32)
    sc_loss = _sc_call(output, tgt)
    if _CUT:
        tc_loss = _tc_call(tgt, output).reshape(-1)
        return jnp.concatenate([tc_loss, sc_loss])
    return sc_loss


# speedup vs baseline: 2.1633x; 2.1633x over previous
import math

import jax
import jax.numpy as jnp
from jax import lax
from jax.experimental import pallas as pl
from jax.experimental.pallas import tpu as pltpu
from jax.experimental.pallas import tpu_sc as plsc

_B = 1024
_V = 100000
_LS = 0.1
_PAD = 0
_CONF = 1.0 - _LS
_SMOOTH = _LS / (_V - 2)
_CONST = _LS * math.log(_SMOOTH) + _CONF * math.log(_CONF)

_CUT = 0
_BB = 32
_NC = 2
_NSUB = 16
_NW = _NC * _NSUB
_RPW = (_B - _CUT) // _NW
_NGRP = _RPW // 8
_TAIL0 = 99968
_TAILW = _V - _TAIL0
_NCHD = 13
_TPCF = 63
_TPCL = 25
_CWF = _TPCF * 128
_CWL = _TPCL * 128 + 32


def _tc_kernel(tgt_ref, x_ref, loss_ref):
    i = pl.program_id(0)
    x = x_ref[...]
    row_sum = jnp.sum(x, axis=1, keepdims=True)

    lane = lax.broadcasted_iota(jnp.int32, (1, 128), 1)
    sel_rows = []
    t_rows = []
    for r in range(_BB):
        t_r = tgt_ref[i * _BB + r]
        base = (t_r // 128) * 128
        chunk = x_ref[r:r + 1, pl.ds(base, 128)]
        sel_rows.append(jnp.where(lane == t_r - base, chunk, 0.0))
        t_rows.append(jnp.full((1, 1), t_r, dtype=jnp.int32))
    o_t = jnp.sum(jnp.concatenate(sel_rows, axis=0), axis=1, keepdims=True)
    t_vec = jnp.concatenate(t_rows, axis=0)

    o_pad = x[:, _PAD:_PAD + 1]
    loss = _SMOOTH * row_sum + (_CONF - _SMOOTH) * o_t - _SMOOTH * o_pad - _CONST
    loss_ref[...] = jnp.where(t_vec == _PAD, 0.0, loss)


def _tc_call(tgt, output):
    grid_spec = pltpu.PrefetchScalarGridSpec(
        num_scalar_prefetch=1,
        grid=(_CUT // _BB,),
        in_specs=[pl.BlockSpec((_BB, _V), lambda i, t: (i, 0))],
        out_specs=pl.BlockSpec((_BB, 1), lambda i, t: (i, 0)),
    )
    return pl.pallas_call(
        _tc_kernel,
        grid_spec=grid_spec,
        out_shape=jax.ShapeDtypeStruct((_CUT, 1), jnp.float32),
    )(tgt, output)


def _sc_body(out_ref, tgt_ref, loss_ref,
             buf0, buf1, tgtv, otv, opv, matv, lossv, sem0, sem1):
    wid = lax.axis_index("s") * _NC + lax.axis_index("c")
    row0 = _CUT + wid * _RPW
    lane = lax.iota(jnp.int32, 16)
    zero16f = jnp.zeros((16,), jnp.float32)
    zero16i = jnp.zeros((16,), jnp.int32)

    pltpu.sync_copy(tgt_ref.at[pl.ds(row0, _RPW)], tgtv)
    for hh in range(_RPW // 16):
        otv[pl.ds(hh * 16, 16)] = zero16f
        opv[pl.ds(hh * 16, 16)] = zero16f

    bufs = (buf0, buf1)
    sems = (sem0, sem1)
    ntot = _NGRP * _NCHD

    def _dma(j, b):
        g = j // _NCHD
        c = j - g * _NCHD
        r8 = row0 + g * 8
        full = pltpu.make_async_copy(
            out_ref.at[pl.ds(r8, 8), pl.ds(c * _CWF, _CWF)], bufs[b], sems[b])
        small = pltpu.make_async_copy(
            out_ref.at[pl.ds(r8, 8), pl.ds((_NCHD - 1) * _CWF, _CWL)],
            bufs[b].at[:, pl.ds(0, _CWL)], sems[b])
        return full, small, c == _NCHD - 1

    def _fire(j, b):
        full, small, is_last = _dma(j, b)

        @pl.when(is_last)
        def _():
            small.start()

        @pl.when(jnp.logical_not(is_last))
        def _():
            full.start()

    def _wait(j, b):
        full, small, is_last = _dma(j, b)

        @pl.when(is_last)
        def _():
            small.wait()

        @pl.when(jnp.logical_not(is_last))
        def _():
            full.wait()

    _fire(0, 0)
    _fire(1, 1)

    zeros8 = tuple(zero16f for _ in range(8))

    def pair_body(jj, accs):
        for b in range(2):
            j = jj * 2 + b
            g = j // _NCHD
            c = j - g * _NCHD
            is_last = c == _NCHD - 1
            _wait(j, b)
            buf = bufs[b]
            nm = jnp.where(is_last, _TPCL, _TPCF)

            def mbody(m, accs):
                out = []
                for a in range(8):
                    acc = accs[a]
                    for l0 in range(8):
                        acc = acc + buf[a, pl.ds(m * 128 + l0 * 16, 16)]
                    out.append(acc)
                return tuple(out)

            accs = lax.fori_loop(0, nm, mbody, accs)
            accs = tuple(
                a + jnp.where(is_last,
                              buf[i, pl.ds(_TPCL * 128, 16)]
                              + buf[i, pl.ds(_TPCL * 128 + 16, 16)], 0.0)
                for i, a in enumerate(accs))

            gm = g - (g // 2) * 2
            hoff = (g // 2) * 16
            t16 = tgtv[pl.ds(hoff, 16)]
            rel = t16 - c * _CWF
            cw = jnp.where(is_last, _CWL, _CWF)
            in_grp = (lane >= gm * 8) & (lane < gm * 8 + 8)
            cond = in_grp & (rel >= 0) & (rel < cw)
            a16 = jnp.clip(lane - gm * 8, 0, 7)
            c16 = jnp.clip(rel, 0, cw - 1)
            gt = plsc.load_gather(buf, [a16, c16])
            otv[pl.ds(hoff, 16)] = jnp.where(cond, gt, otv[pl.ds(hoff, 16)])
            gp = plsc.load_gather(buf, [a16, zero16i])
            opv[pl.ds(hoff, 16)] = jnp.where(
                in_grp & (c == 0), gp, opv[pl.ds(hoff, 16)])

            @pl.when(j + 2 < ntot)
            def _():
                _fire(j + 2, b)

            @pl.when(is_last)
            def _():
                for a in range(8):
                    matv[pl.ds((g * 8 + a) * 16, 16)] = accs[a]

            accs = tuple(jnp.where(is_last, 0.0, a) for a in accs)
        return accs

    lax.fori_loop(0, ntot // 2, pair_body, zeros8)

    for hh in range(_RPW // 16):
        s16 = zero16f
        for q in range(16):
            s16 = s16 + plsc.load_gather(matv, [(hh * 16 + lane) * 16 + q])
        t16 = tgtv[pl.ds(hh * 16, 16)]
        loss = (_SMOOTH * s16 + (_CONF - _SMOOTH) * otv[pl.ds(hh * 16, 16)]
                - _SMOOTH * opv[pl.ds(hh * 16, 16)] - _CONST)
        lossv[pl.ds(hh * 16, 16)] = jnp.where(t16 == _PAD, 0.0, loss)
    pltpu.sync_copy(lossv, loss_ref.at[pl.ds(wid * _RPW, _RPW)])


def _sc_call(output, tgt):
    mesh = plsc.VectorSubcoreMesh(core_axis_name="c", subcore_axis_name="s")
    fn = pl.kernel(
        _sc_body,
        out_type=jax.ShapeDtypeStruct((_B - _CUT,), jnp.float32),
        mesh=mesh,
        scratch_types=[
            pltpu.VMEM((8, _CWF), jnp.float32),
            pltpu.VMEM((8, _CWF), jnp.float32),
            pltpu.VMEM((_RPW,), jnp.int32),
            pltpu.VMEM((_RPW,), jnp.float32),
            pltpu.VMEM((_RPW,), jnp.float32),
            pltpu.VMEM((_RPW * 16,), jnp.float32),
            pltpu.VMEM((_RPW,), jnp.float32),
            pltpu.SemaphoreType.DMA,
            pltpu.SemaphoreType.DMA,
        ],
        compiler_params=pltpu.CompilerParams(
            needs_layout_passes=False, use_tc_tiling_on_sc=False),
    )
    return fn(output, tgt)


_ABB = 32
_ANB = _B // _ABB


def _tc_any_kernel(tgt_ref, out_hbm, loss_ref, vb0, vb1, s0, s1):
    vbs = (vb0, vb1)
    sems = (s0, s1)
    lane = lax.broadcasted_iota(jnp.int32, (1, 128), 1)

    def _cp(j, b):
        return pltpu.make_async_copy(
            out_hbm.at[pl.ds(j * _ABB, _ABB), :], vbs[b], sems[b])

    _cp(0, 0).start()
    for j in range(_ANB):
        b = j % 2
        if j + 1 < _ANB:
            _cp(j + 1, 1 - b).start()
        _cp(j, b).wait()
        x = vbs[b][...]
        row_sum = jnp.sum(x, axis=1, keepdims=True)
        sel_rows = []
        t_rows = []
        for r in range(_ABB):
            t_r = tgt_ref[j * _ABB + r]
            base = (t_r // 128) * 128
            chunk = vbs[b][r:r + 1, pl.ds(base, 128)]
            sel_rows.append(jnp.where(lane == t_r - base, chunk, 0.0))
            t_rows.append(jnp.full((1, 1), t_r, dtype=jnp.int32))
        o_t = jnp.sum(jnp.concatenate(sel_rows, axis=0), axis=1, keepdims=True)
        t_vec = jnp.concatenate(t_rows, axis=0)
        o_pad = x[:, _PAD:_PAD + 1]
        loss = (_SMOOTH * row_sum + (_CONF - _SMOOTH) * o_t
                - _SMOOTH * o_pad - _CONST)
        loss_ref[pl.ds(j * _ABB, _ABB), :] = jnp.where(t_vec == _PAD, 0.0, loss)


def _tc_any_call(tgt, output):
    grid_spec = pltpu.PrefetchScalarGridSpec(
        num_scalar_prefetch=1,
        grid=(1,),
        in_specs=[pl.BlockSpec(memory_space=pl.ANY)],
        out_specs=pl.BlockSpec((_B, 1), lambda i, t: (0, 0)),
        scratch_shapes=[
            pltpu.VMEM((_ABB, _V), jnp.float32),
            pltpu.VMEM((_ABB, _V), jnp.float32),
            pltpu.SemaphoreType.DMA,
            pltpu.SemaphoreType.DMA,
        ],
    )
    return pl.pallas_call(
        _tc_any_kernel,
        grid_spec=grid_spec,
        out_shape=jax.ShapeDtypeStruct((_B, 1), jnp.float32),
    )(tgt, output)


def kernel(output, target):
    tgt = target.astype(jnp.int32)
    return _tc_any_call(tgt, output).reshape(-1)
